# baseline (speedup 1.0000x reference)
import functools

import jax
import jax.numpy as jnp
from jax import lax
from jax.experimental import pallas as pl
from jax.experimental.pallas import tpu as pltpu
from jax.experimental.pallas import tpu_sc as plsc

_N = 10000
_E = 320000
_F = 128
_C = 50
_NW = 32
_NCH = _E // (_NW * _C)
_NSUB = 16
_RPS = 624
_RPS_LAST = _N - 15 * _RPS
_BM = 1000


_K = 4
_ST = 5
_SCH = _NCH // _ST


def _make_segsum(with_counts: bool):
    mesh = plsc.VectorSubcoreMesh(core_axis_name="c", subcore_axis_name="s")
    out_type = [jax.ShapeDtypeStruct((2, _N, _F), jnp.float32)]
    scratch = [
        pltpu.VMEM((_SCH, _C), jnp.int32),
        pltpu.VMEM((_SCH, _C), jnp.int32),
        [pltpu.VMEM((_C, _F), jnp.float32) for _ in range(_K)],
        pltpu.VMEM_SHARED((_N, _F), jnp.float32),
        [pltpu.SemaphoreType.DMA for _ in range(_K)],
        [pltpu.SemaphoreType.DMA for _ in range(_K)],
    ]
    if with_counts:
        out_type.append(jax.ShapeDtypeStruct((2, _N), jnp.float32))
        scratch += [
            pltpu.VMEM((_C,), jnp.float32),
            pltpu.VMEM_SHARED((_N,), jnp.float32),
            [pltpu.SemaphoreType.DMA for _ in range(_K)],
        ]

    def body(y, src_i, dst_i, zrow, zc, ones, parts, counts,
             src_v, dst_v, rows, acc, gsem, ssem,
             ones_v=None, cacc=None, csem=None):
        c = lax.axis_index("c")
        s = lax.axis_index("s")
        wid = c * _NSUB + s
        @pl.when(s < 15)
        def _():
            pltpu.sync_copy(zrow.at[pl.ds(0, _RPS)],
                            acc.at[pl.ds(s * _RPS, _RPS)])

        @pl.when(s == 15)
        def _():
            pltpu.sync_copy(zrow, acc.at[pl.ds(15 * _RPS, _RPS_LAST)])
        if with_counts:
            @pl.when(s == 0)
            def _():
                pltpu.sync_copy(zc, cacc)
            pltpu.sync_copy(ones, ones_v)
        plsc.subcore_barrier()

        def g_start(b, j):
            pltpu.async_copy(y.at[src_v.at[j]], rows[b], gsem[b])

        def g_wait(b):
            pltpu.make_async_copy(y.at[src_v.at[0]], rows[b], gsem[b]).wait()

        def s_start(b, j):
            pltpu.async_copy(rows[b], acc.at[dst_v.at[j]], ssem[b], add=True)
            if with_counts:
                pltpu.async_copy(ones_v, cacc.at[dst_v.at[j]], csem[b],
                                 add=True)

        def s_wait(b):
            pltpu.make_async_copy(rows[b], acc.at[dst_v.at[0]],
                                  ssem[b]).wait()
            if with_counts:
                pltpu.make_async_copy(ones_v, cacc.at[dst_v.at[0]],
                                      csem[b]).wait()

        def stage(t, carry):
            pltpu.sync_copy(src_i.at[wid, pl.ds(t * _SCH, _SCH)], src_v)
            pltpu.sync_copy(dst_i.at[wid, pl.ds(t * _SCH, _SCH)], dst_v)
            g_start(0, 0)
            g_start(1, 1)

            def quad(i, c2):
                for u in range(_K):
                    j = i * _K + u
                    g_wait(u)
                    s_start(u, j)
                    d = (u + 2) % _K
                    if u < 2:
                        @pl.when(i > 0)
                        def _():
                            s_wait(d)
                        g_start(d, j + 2)
                    else:
                        s_wait(d)

                        @pl.when(i + 1 < _SCH // _K)
                        def _():
                            g_start(d, j + 2)
                return c2

            lax.fori_loop(0, _SCH // _K, quad, 0)
            s_wait(2)
            s_wait(3)
            return carry

        lax.fori_loop(0, _ST, stage, 0)
        plsc.subcore_barrier()

        @pl.when(s < 15)
        def _():
            pltpu.sync_copy(acc.at[pl.ds(s * _RPS, _RPS)],
                            parts.at[c, pl.ds(s * _RPS, _RPS)])

        @pl.when(s == 15)
        def _():
            pltpu.sync_copy(acc.at[pl.ds(15 * _RPS, _RPS_LAST)],
                            parts.at[c, pl.ds(15 * _RPS, _RPS_LAST)])
        if with_counts:
            @pl.when(s == 0)
            def _():
                pltpu.sync_copy(cacc, counts.at[c])

    if with_counts:
        def body_wc(y, src_i, dst_i, zrow, zc, ones, parts, counts,
                    src_v, dst_v, rows, acc, gsem, ssem, ones_v, cacc, csem):
            body(y, src_i, dst_i, zrow, zc, ones, parts, counts,
                 src_v, dst_v, rows, acc, gsem, ssem, ones_v, cacc, csem)
        fn = body_wc
    else:
        def body_nc(y, src_i, dst_i, zrow, zc, ones, parts,
                    src_v, dst_v, rows, acc, gsem, ssem):
            body(y, src_i, dst_i, zrow, zc, ones, parts, None,
                 src_v, dst_v, rows, acc, gsem, ssem)
        fn = body_nc

    return pl.kernel(fn, mesh=mesh, out_type=out_type, scratch_types=scratch)


_segsum_counts = _make_segsum(True)
_segsum = _make_segsum(False)


def _mm2_body(x_ref, wl_ref, wr_ref, b_ref, y_ref, r_ref):
    xb = x_ref[...]
    y_ref[...] = jnp.dot(xb, wl_ref[...], preferred_element_type=jnp.float32)
    r_ref[...] = (jnp.dot(xb, wr_ref[...], preferred_element_type=jnp.float32)
                  + b_ref[...])


_mm2 = pl.pallas_call(
    _mm2_body,
    grid=(_N // _BM,),
    in_specs=[
        pl.BlockSpec((_BM, _F), lambda i: (i, 0)),
        pl.BlockSpec((_F, _F), lambda i: (0, 0)),
        pl.BlockSpec((_F, _F), lambda i: (0, 0)),
        pl.BlockSpec((1, _F), lambda i: (0, 0)),
    ],
    out_specs=[pl.BlockSpec((_BM, _F), lambda i: (i, 0))] * 2,
    out_shape=[jax.ShapeDtypeStruct((_N, _F), jnp.float32)] * 2,
)


def _comb1_body(p_ref, c_ref, r1_ref, wl_ref, wr_ref, b_ref,
                y2_ref, r2_ref, inv_ref):
    p = p_ref[0] + p_ref[1]
    cnt = c_ref[0] + c_ref[1]
    inv = 1.0 / jnp.maximum(cnt, 1.0)
    h = jnp.maximum(p * inv + r1_ref[...], 0.0)
    y2_ref[...] = jnp.dot(h, wl_ref[...], preferred_element_type=jnp.float32)
    r2_ref[...] = (jnp.dot(h, wr_ref[...], preferred_element_type=jnp.float32)
                   + b_ref[...])
    inv_ref[...] = inv


_comb1 = pl.pallas_call(
    _comb1_body,
    grid=(_N // _BM,),
    in_specs=[
        pl.BlockSpec((2, _BM, _F), lambda i: (0, i, 0)),
        pl.BlockSpec((2, _BM, 1), lambda i: (0, i, 0)),
        pl.BlockSpec((_BM, _F), lambda i: (i, 0)),
        pl.BlockSpec((_F, _F), lambda i: (0, 0)),
        pl.BlockSpec((_F, _F), lambda i: (0, 0)),
        pl.BlockSpec((1, _F), lambda i: (0, 0)),
    ],
    out_specs=[
        pl.BlockSpec((_BM, _F), lambda i: (i, 0)),
        pl.BlockSpec((_BM, _F), lambda i: (i, 0)),
        pl.BlockSpec((_BM, 1), lambda i: (i, 0)),
    ],
    out_shape=[
        jax.ShapeDtypeStruct((_N, _F), jnp.float32),
        jax.ShapeDtypeStruct((_N, _F), jnp.float32),
        jax.ShapeDtypeStruct((_N, 1), jnp.float32),
    ],
)


def _comb2_body(p_ref, inv_ref, r2_ref, o_ref):
    p = p_ref[0] + p_ref[1]
    o_ref[...] = p * inv_ref[...] + r2_ref[...]


_comb2 = pl.pallas_call(
    _comb2_body,
    grid=(_N // _BM,),
    in_specs=[
        pl.BlockSpec((2, _BM, _F), lambda i: (0, i, 0)),
        pl.BlockSpec((_BM, 1), lambda i: (i, 0)),
        pl.BlockSpec((_BM, _F), lambda i: (i, 0)),
    ],
    out_specs=pl.BlockSpec((_BM, _F), lambda i: (i, 0)),
    out_shape=jax.ShapeDtypeStruct((_N, _F), jnp.float32),
)


def kernel(x, edge_index, edge_attr, W1_l, b1, W1_r, W2_l, b2, W2_r):
    del edge_attr
    src = edge_index[0].reshape(_NW, _NCH, _C).astype(jnp.int32)
    dst = edge_index[1].reshape(_NW, _NCH, _C).astype(jnp.int32)
    zrow = jnp.zeros((_RPS_LAST, _F), jnp.float32)
    zc = jnp.zeros((_N,), jnp.float32)
    ones = jnp.ones((_C,), jnp.float32)

    y1, r1 = _mm2(x, W1_l, W1_r, b1.reshape(1, _F))
    parts1, cnts = _segsum_counts(y1, src, dst, zrow, zc, ones)
    y2, r2, inv = _comb1(parts1, cnts.reshape(2, _N, 1), r1,
                         W2_l, W2_r, b2.reshape(1, _F))
    parts2, = _segsum(y2, src, dst, zrow, zc, ones)
    out = _comb2(parts2, inv, r2)
    return out


# speedup vs baseline: 1.1129x; 1.1129x over previous
import functools

import jax
import jax.numpy as jnp
from jax import lax
from jax.experimental import pallas as pl
from jax.experimental.pallas import tpu as pltpu
from jax.experimental.pallas import tpu_sc as plsc

_N = 10000
_E = 320000
_F = 128
_C = 40
_NW = 32
_NCH = _E // (_NW * _C)
_NSUB = 16
_RPS = 624
_RPS_LAST = _N - 15 * _RPS
_BM = 1000


_K = 5
_ST = 5
_SCH = _NCH // _ST


def _make_segsum(with_counts: bool):
    mesh = plsc.VectorSubcoreMesh(core_axis_name="c", subcore_axis_name="s")
    out_type = [jax.ShapeDtypeStruct((2, _N, _F), jnp.float32)]
    scratch = [
        pltpu.VMEM((_SCH, _C), jnp.int32),
        pltpu.VMEM((_SCH, _C), jnp.int32),
        [pltpu.VMEM((_C, _F), jnp.float32) for _ in range(_K)],
        pltpu.VMEM_SHARED((_N, _F), jnp.float32),
        [pltpu.SemaphoreType.DMA for _ in range(_K)],
        [pltpu.SemaphoreType.DMA for _ in range(_K)],
    ]
    if with_counts:
        out_type.append(jax.ShapeDtypeStruct((2, _N), jnp.float32))
        scratch += [
            pltpu.VMEM((_C,), jnp.float32),
            pltpu.VMEM_SHARED((_N,), jnp.float32),
            [pltpu.SemaphoreType.DMA for _ in range(_K)],
        ]

    def body(y, src_i, dst_i, zrow, zc, ones, parts, counts,
             src_v, dst_v, rows, acc, gsem, ssem,
             ones_v=None, cacc=None, csem=None):
        c = lax.axis_index("c")
        s = lax.axis_index("s")
        wid = c * _NSUB + s
        @pl.when(s < 15)
        def _():
            pltpu.sync_copy(zrow.at[pl.ds(0, _RPS)],
                            acc.at[pl.ds(s * _RPS, _RPS)])

        @pl.when(s == 15)
        def _():
            pltpu.sync_copy(zrow, acc.at[pl.ds(15 * _RPS, _RPS_LAST)])
        if with_counts:
            @pl.when(s == 0)
            def _():
                pltpu.sync_copy(zc, cacc)
            pltpu.sync_copy(ones, ones_v)
        plsc.subcore_barrier()

        def g_start(b, j):
            pltpu.async_copy(y.at[src_v.at[j]], rows[b], gsem[b])

        def g_wait(b):
            pltpu.make_async_copy(y.at[src_v.at[0]], rows[b], gsem[b]).wait()

        def s_start(b, j):
            pltpu.async_copy(rows[b], acc.at[dst_v.at[j]], ssem[b], add=True)
            if with_counts:
                pltpu.async_copy(ones_v, cacc.at[dst_v.at[j]], csem[b],
                                 add=True)

        def s_wait(b):
            pltpu.make_async_copy(rows[b], acc.at[dst_v.at[0]],
                                  ssem[b]).wait()
            if with_counts:
                pltpu.make_async_copy(ones_v, cacc.at[dst_v.at[0]],
                                      csem[b]).wait()

        def stage(t, carry):
            pltpu.sync_copy(src_i.at[wid, t], src_v)
            pltpu.sync_copy(dst_i.at[wid, t], dst_v)
            for b in range(3):
                g_start(b, b)

            def turn(i, c2):
                for u in range(_K):
                    j = i * _K + u
                    g_wait(u)
                    s_start(u, j)
                    d = (u + 3) % _K
                    if u < 2:
                        @pl.when(i > 0)
                        def _():
                            s_wait(d)
                        g_start(d, j + 3)
                    else:
                        s_wait(d)

                        @pl.when(i + 1 < _SCH // _K)
                        def _():
                            g_start(d, j + 3)
                return c2

            lax.fori_loop(0, _SCH // _K, turn, 0)
            s_wait(3)
            s_wait(4)
            return carry

        lax.fori_loop(0, _ST, stage, 0)
        plsc.subcore_barrier()

        @pl.when(s < 15)
        def _():
            pltpu.sync_copy(acc.at[pl.ds(s * _RPS, _RPS)],
                            parts.at[c, pl.ds(s * _RPS, _RPS)])

        @pl.when(s == 15)
        def _():
            pltpu.sync_copy(acc.at[pl.ds(15 * _RPS, _RPS_LAST)],
                            parts.at[c, pl.ds(15 * _RPS, _RPS_LAST)])
        if with_counts:
            @pl.when(s == 0)
            def _():
                pltpu.sync_copy(cacc, counts.at[c])

    if with_counts:
        def body_wc(y, src_i, dst_i, zrow, zc, ones, parts, counts,
                    src_v, dst_v, rows, acc, gsem, ssem, ones_v, cacc, csem):
            body(y, src_i, dst_i, zrow, zc, ones, parts, counts,
                 src_v, dst_v, rows, acc, gsem, ssem, ones_v, cacc, csem)
        fn = body_wc
    else:
        def body_nc(y, src_i, dst_i, zrow, zc, ones, parts,
                    src_v, dst_v, rows, acc, gsem, ssem):
            body(y, src_i, dst_i, zrow, zc, ones, parts, None,
                 src_v, dst_v, rows, acc, gsem, ssem)
        fn = body_nc

    return pl.kernel(fn, mesh=mesh, out_type=out_type, scratch_types=scratch)


_segsum_counts = _make_segsum(True)
_segsum = _make_segsum(False)


def _mm2_body(x_ref, wl_ref, wr_ref, b_ref, y_ref, r_ref):
    xb = x_ref[...]
    y_ref[...] = jnp.dot(xb, wl_ref[...], preferred_element_type=jnp.float32)
    r_ref[...] = (jnp.dot(xb, wr_ref[...], preferred_element_type=jnp.float32)
                  + b_ref[...])


_mm2 = pl.pallas_call(
    _mm2_body,
    grid=(_N // _BM,),
    in_specs=[
        pl.BlockSpec((_BM, _F), lambda i: (i, 0)),
        pl.BlockSpec((_F, _F), lambda i: (0, 0)),
        pl.BlockSpec((_F, _F), lambda i: (0, 0)),
        pl.BlockSpec((1, _F), lambda i: (0, 0)),
    ],
    out_specs=[pl.BlockSpec((_BM, _F), lambda i: (i, 0))] * 2,
    out_shape=[jax.ShapeDtypeStruct((_N, _F), jnp.float32)] * 2,
)


def _comb1_body(p_ref, c_ref, r1_ref, wl_ref, wr_ref, b_ref,
                y2_ref, r2_ref, inv_ref):
    p = p_ref[0] + p_ref[1]
    cnt = c_ref[0] + c_ref[1]
    inv = 1.0 / jnp.maximum(cnt, 1.0)
    h = jnp.maximum(p * inv + r1_ref[...], 0.0)
    y2_ref[...] = jnp.dot(h, wl_ref[...], preferred_element_type=jnp.float32)
    r2_ref[...] = (jnp.dot(h, wr_ref[...], preferred_element_type=jnp.float32)
                   + b_ref[...])
    inv_ref[...] = inv


_comb1 = pl.pallas_call(
    _comb1_body,
    grid=(_N // _BM,),
    in_specs=[
        pl.BlockSpec((2, _BM, _F), lambda i: (0, i, 0)),
        pl.BlockSpec((2, _BM, 1), lambda i: (0, i, 0)),
        pl.BlockSpec((_BM, _F), lambda i: (i, 0)),
        pl.BlockSpec((_F, _F), lambda i: (0, 0)),
        pl.BlockSpec((_F, _F), lambda i: (0, 0)),
        pl.BlockSpec((1, _F), lambda i: (0, 0)),
    ],
    out_specs=[
        pl.BlockSpec((_BM, _F), lambda i: (i, 0)),
        pl.BlockSpec((_BM, _F), lambda i: (i, 0)),
        pl.BlockSpec((_BM, 1), lambda i: (i, 0)),
    ],
    out_shape=[
        jax.ShapeDtypeStruct((_N, _F), jnp.float32),
        jax.ShapeDtypeStruct((_N, _F), jnp.float32),
        jax.ShapeDtypeStruct((_N, 1), jnp.float32),
    ],
)


def _comb2_body(p_ref, inv_ref, r2_ref, o_ref):
    p = p_ref[0] + p_ref[1]
    o_ref[...] = p * inv_ref[...] + r2_ref[...]


_comb2 = pl.pallas_call(
    _comb2_body,
    grid=(_N // _BM,),
    in_specs=[
        pl.BlockSpec((2, _BM, _F), lambda i: (0, i, 0)),
        pl.BlockSpec((_BM, 1), lambda i: (i, 0)),
        pl.BlockSpec((_BM, _F), lambda i: (i, 0)),
    ],
    out_specs=pl.BlockSpec((_BM, _F), lambda i: (i, 0)),
    out_shape=jax.ShapeDtypeStruct((_N, _F), jnp.float32),
)


def kernel(x, edge_index, edge_attr, W1_l, b1, W1_r, W2_l, b2, W2_r):
    del edge_attr
    src = edge_index[0].reshape(_NW, _ST, _SCH, _C).astype(jnp.int32)
    dst = edge_index[1].reshape(_NW, _ST, _SCH, _C).astype(jnp.int32)
    zrow = jnp.zeros((_RPS_LAST, _F), jnp.float32)
    zc = jnp.zeros((_N,), jnp.float32)
    ones = jnp.ones((_C,), jnp.float32)

    y1, r1 = _mm2(x, W1_l, W1_r, b1.reshape(1, _F))
    parts1, cnts = _segsum_counts(y1, src, dst, zrow, zc, ones)
    y2, r2, inv = _comb1(parts1, cnts.reshape(2, _N, 1), r1,
                         W2_l, W2_r, b2.reshape(1, _F))
    parts2, = _segsum(y2, src, dst, zrow, zc, ones)
    out = _comb2(parts2, inv, r2)
    return out


# speedup vs baseline: 1.1634x; 1.0454x over previous
import functools

import jax
import jax.numpy as jnp
from jax import lax
from jax.experimental import pallas as pl
from jax.experimental.pallas import tpu as pltpu
from jax.experimental.pallas import tpu_sc as plsc

_N = 10000
_E = 320000
_F = 128
_C = 40
_NW = 32
_NCH = _E // (_NW * _C)
_NSUB = 16
_RPS = 624
_RPS_LAST = _N - 15 * _RPS
_BM = 1000


_G = 3
_L = 2
_K = _G + _L
_ST = 10
_SCH = _NCH // _ST
_NT = _SCH // _K


def _make_segsum(with_counts: bool):
    mesh = plsc.VectorSubcoreMesh(core_axis_name="c", subcore_axis_name="s")
    out_type = [jax.ShapeDtypeStruct((2, _N, _F), jnp.float32)]
    scratch = [
        [pltpu.VMEM((_SCH, _C), jnp.int32) for _ in range(2)],
        [pltpu.VMEM((_SCH, _C), jnp.int32) for _ in range(2)],
        [pltpu.VMEM((_C, _F), jnp.float32) for _ in range(_K)],
        pltpu.VMEM_SHARED((_N, _F), jnp.float32),
        [pltpu.SemaphoreType.DMA for _ in range(_K)],
        [pltpu.SemaphoreType.DMA for _ in range(_K)],
        pltpu.SemaphoreType.DMA,
    ]
    if with_counts:
        out_type.append(jax.ShapeDtypeStruct((2, _N), jnp.float32))
        scratch += [
            pltpu.VMEM((_C,), jnp.float32),
            pltpu.VMEM_SHARED((_N,), jnp.float32),
            [pltpu.SemaphoreType.DMA for _ in range(_K)],
        ]

    def body(y, src_i, dst_i, zrow, zc, ones, parts, counts,
             src_v, dst_v, rows, acc, gsem, ssem, isem,
             ones_v=None, cacc=None, csem=None):
        c = lax.axis_index("c")
        s = lax.axis_index("s")
        wid = c * _NSUB + s
        @pl.when(s < 15)
        def _():
            pltpu.sync_copy(zrow.at[pl.ds(0, _RPS)],
                            acc.at[pl.ds(s * _RPS, _RPS)])

        @pl.when(s == 15)
        def _():
            pltpu.sync_copy(zrow, acc.at[pl.ds(15 * _RPS, _RPS_LAST)])
        if with_counts:
            @pl.when(s == 0)
            def _():
                pltpu.sync_copy(zc, cacc)
            pltpu.sync_copy(ones, ones_v)
        plsc.subcore_barrier()

        def g_start(b, sb, j):
            pltpu.async_copy(y.at[sb.at[j]], rows[b], gsem[b])

        def g_wait(b):
            pltpu.make_async_copy(y.at[src_v[0].at[0]], rows[b],
                                  gsem[b]).wait()

        def s_start(b, db, j):
            pltpu.async_copy(rows[b], acc.at[db.at[j]], ssem[b], add=True)
            if with_counts:
                pltpu.async_copy(ones_v, cacc.at[db.at[j]], csem[b],
                                 add=True)

        def s_wait(b):
            pltpu.make_async_copy(rows[b], acc.at[dst_v[0].at[0]],
                                  ssem[b]).wait()
            if with_counts:
                pltpu.make_async_copy(ones_v, cacc.at[dst_v[0].at[0]],
                                      csem[b]).wait()

        def i_start(t_next, nsb, ndb):
            pltpu.async_copy(src_i.at[wid, t_next], nsb, isem)
            pltpu.async_copy(dst_i.at[wid, t_next], ndb, isem)

        def i_wait(nsb, ndb):
            pltpu.make_async_copy(src_i.at[wid, 0], nsb, isem).wait()
            pltpu.make_async_copy(dst_i.at[wid, 0], ndb, isem).wait()

        def phase(sb, db, nsb, ndb, t_next, pre, sw0):
            if pre is None:
                i_start(t_next, nsb, ndb)
            else:
                @pl.when(pre)
                def _():
                    i_start(t_next, nsb, ndb)

            def turn_reg(i, c2):
                for u in range(_K):
                    j = i * _K + u
                    g_wait(u)
                    s_start(u, db, j)
                    d = (u + _G) % _K
                    if u < _L and sw0 is not None:
                        @pl.when(sw0 | (i > 0))
                        def _():
                            s_wait(d)
                    else:
                        s_wait(d)
                    g_start(d, sb, j + _G)
                return c2

            lax.fori_loop(0, _NT - 1, turn_reg, 0)
            for u in range(_K):
                j = (_NT - 1) * _K + u
                g_wait(u)
                s_start(u, db, j)
                d = (u + _G) % _K
                s_wait(d)
                if u < _L:
                    g_start(d, sb, j + _G)
                else:
                    if u == _L:
                        if pre is None:
                            i_wait(nsb, ndb)
                        else:
                            @pl.when(pre)
                            def _():
                                i_wait(nsb, ndb)
                    if pre is None:
                        g_start(d, nsb, j + _G - _SCH)
                    else:
                        @pl.when(pre)
                        def _():
                            g_start(d, nsb, j + _G - _SCH)

        pltpu.sync_copy(src_i.at[wid, 0], src_v[0])
        pltpu.sync_copy(dst_i.at[wid, 0], dst_v[0])
        for b in range(_G):
            g_start(b, src_v[0], b)

        def pair(p, carry):
            phase(src_v[0], dst_v[0], src_v[1], dst_v[1],
                  2 * p + 1, None, p > 0)
            phase(src_v[1], dst_v[1], src_v[0], dst_v[0],
                  2 * p + 2, p + 1 < _ST // 2, None)
            return carry

        lax.fori_loop(0, _ST // 2, pair, 0)
        for b in range(_K - _L, _K):
            s_wait(b)
        plsc.subcore_barrier()

        @pl.when(s < 15)
        def _():
            pltpu.sync_copy(acc.at[pl.ds(s * _RPS, _RPS)],
                            parts.at[c, pl.ds(s * _RPS, _RPS)])

        @pl.when(s == 15)
        def _():
            pltpu.sync_copy(acc.at[pl.ds(15 * _RPS, _RPS_LAST)],
                            parts.at[c, pl.ds(15 * _RPS, _RPS_LAST)])
        if with_counts:
            @pl.when(s == 0)
            def _():
                pltpu.sync_copy(cacc, counts.at[c])

    if with_counts:
        def body_wc(y, src_i, dst_i, zrow, zc, ones, parts, counts,
                    src_v, dst_v, rows, acc, gsem, ssem, isem,
                    ones_v, cacc, csem):
            body(y, src_i, dst_i, zrow, zc, ones, parts, counts,
                 src_v, dst_v, rows, acc, gsem, ssem, isem,
                 ones_v, cacc, csem)
        fn = body_wc
    else:
        def body_nc(y, src_i, dst_i, zrow, zc, ones, parts,
                    src_v, dst_v, rows, acc, gsem, ssem, isem):
            body(y, src_i, dst_i, zrow, zc, ones, parts, None,
                 src_v, dst_v, rows, acc, gsem, ssem, isem)
        fn = body_nc

    return pl.kernel(fn, mesh=mesh, out_type=out_type, scratch_types=scratch)


_segsum_counts = _make_segsum(True)
_segsum = _make_segsum(False)


def _mm2_body(x_ref, wl_ref, wr_ref, b_ref, y_ref, r_ref):
    xb = x_ref[...]
    y_ref[...] = jnp.dot(xb, wl_ref[...], preferred_element_type=jnp.float32)
    r_ref[...] = (jnp.dot(xb, wr_ref[...], preferred_element_type=jnp.float32)
                  + b_ref[...])


_mm2 = pl.pallas_call(
    _mm2_body,
    grid=(_N // _BM,),
    in_specs=[
        pl.BlockSpec((_BM, _F), lambda i: (i, 0)),
        pl.BlockSpec((_F, _F), lambda i: (0, 0)),
        pl.BlockSpec((_F, _F), lambda i: (0, 0)),
        pl.BlockSpec((1, _F), lambda i: (0, 0)),
    ],
    out_specs=[pl.BlockSpec((_BM, _F), lambda i: (i, 0))] * 2,
    out_shape=[jax.ShapeDtypeStruct((_N, _F), jnp.float32)] * 2,
)


def _comb1_body(p_ref, c_ref, r1_ref, wl_ref, wr_ref, b_ref,
                y2_ref, r2_ref, inv_ref):
    p = p_ref[0] + p_ref[1]
    cnt = c_ref[0] + c_ref[1]
    inv = 1.0 / jnp.maximum(cnt, 1.0)
    h = jnp.maximum(p * inv + r1_ref[...], 0.0)
    y2_ref[...] = jnp.dot(h, wl_ref[...], preferred_element_type=jnp.float32)
    r2_ref[...] = (jnp.dot(h, wr_ref[...], preferred_element_type=jnp.float32)
                   + b_ref[...])
    inv_ref[...] = inv


_comb1 = pl.pallas_call(
    _comb1_body,
    grid=(_N // _BM,),
    in_specs=[
        pl.BlockSpec((2, _BM, _F), lambda i: (0, i, 0)),
        pl.BlockSpec((2, _BM, 1), lambda i: (0, i, 0)),
        pl.BlockSpec((_BM, _F), lambda i: (i, 0)),
        pl.BlockSpec((_F, _F), lambda i: (0, 0)),
        pl.BlockSpec((_F, _F), lambda i: (0, 0)),
        pl.BlockSpec((1, _F), lambda i: (0, 0)),
    ],
    out_specs=[
        pl.BlockSpec((_BM, _F), lambda i: (i, 0)),
        pl.BlockSpec((_BM, _F), lambda i: (i, 0)),
        pl.BlockSpec((_BM, 1), lambda i: (i, 0)),
    ],
    out_shape=[
        jax.ShapeDtypeStruct((_N, _F), jnp.float32),
        jax.ShapeDtypeStruct((_N, _F), jnp.float32),
        jax.ShapeDtypeStruct((_N, 1), jnp.float32),
    ],
)


def _comb2_body(p_ref, inv_ref, r2_ref, o_ref):
    p = p_ref[0] + p_ref[1]
    o_ref[...] = p * inv_ref[...] + r2_ref[...]


_comb2 = pl.pallas_call(
    _comb2_body,
    grid=(_N // _BM,),
    in_specs=[
        pl.BlockSpec((2, _BM, _F), lambda i: (0, i, 0)),
        pl.BlockSpec((_BM, 1), lambda i: (i, 0)),
        pl.BlockSpec((_BM, _F), lambda i: (i, 0)),
    ],
    out_specs=pl.BlockSpec((_BM, _F), lambda i: (i, 0)),
    out_shape=jax.ShapeDtypeStruct((_N, _F), jnp.float32),
)


def kernel(x, edge_index, edge_attr, W1_l, b1, W1_r, W2_l, b2, W2_r):
    del edge_attr
    src = edge_index[0].reshape(_NW, _ST, _SCH, _C).astype(jnp.int32)
    dst = edge_index[1].reshape(_NW, _ST, _SCH, _C).astype(jnp.int32)
    zrow = jnp.zeros((_RPS_LAST, _F), jnp.float32)
    zc = jnp.zeros((_N,), jnp.float32)
    ones = jnp.ones((_C,), jnp.float32)

    y1, r1 = _mm2(x, W1_l, W1_r, b1.reshape(1, _F))
    parts1, cnts = _segsum_counts(y1, src, dst, zrow, zc, ones)
    y2, r2, inv = _comb1(parts1, cnts.reshape(2, _N, 1), r1,
                         W2_l, W2_r, b2.reshape(1, _F))
    parts2, = _segsum(y2, src, dst, zrow, zc, ones)
    out = _comb2(parts2, inv, r2)
    return out


# speedup vs baseline: 1.2627x; 1.0853x over previous
import functools

import jax
import jax.numpy as jnp
from jax import lax
from jax.experimental import pallas as pl
from jax.experimental.pallas import tpu as pltpu
from jax.experimental.pallas import tpu_sc as plsc

_N = 10000
_E = 320000
_F = 128
_C = 40
_NW = 32
_NCH = _E // (_NW * _C)
_NSUB = 16
_RPS = 624
_RPS_LAST = _N - 15 * _RPS
_BM = 1000


_G = 4
_L = 1
_K = _G + _L
_ST = 10
_SCH = _NCH // _ST
_NT = _SCH // _K


def _make_segsum(with_counts: bool):
    mesh = plsc.VectorSubcoreMesh(core_axis_name="c", subcore_axis_name="s")
    out_type = [jax.ShapeDtypeStruct((2, _N, _F), jnp.float32)]
    scratch = [
        [pltpu.VMEM((_SCH, _C), jnp.int32) for _ in range(2)],
        [pltpu.VMEM((_SCH, _C), jnp.int32) for _ in range(2)],
        [pltpu.VMEM((_C, _F), jnp.float32) for _ in range(_K)],
        pltpu.VMEM_SHARED((_N, _F), jnp.float32),
        [pltpu.SemaphoreType.DMA for _ in range(_K)],
        [pltpu.SemaphoreType.DMA for _ in range(_K)],
        pltpu.SemaphoreType.DMA,
    ]
    if with_counts:
        out_type.append(jax.ShapeDtypeStruct((2, _N), jnp.float32))
        scratch += [
            pltpu.VMEM((_C,), jnp.float32),
            pltpu.VMEM_SHARED((_N,), jnp.float32),
            [pltpu.SemaphoreType.DMA for _ in range(_K)],
        ]

    def body(y, src_i, dst_i, zrow, zc, ones, parts, counts,
             src_v, dst_v, rows, acc, gsem, ssem, isem,
             ones_v=None, cacc=None, csem=None):
        c = lax.axis_index("c")
        s = lax.axis_index("s")
        wid = c * _NSUB + s
        @pl.when(s < 15)
        def _():
            pltpu.sync_copy(zrow.at[pl.ds(0, _RPS)],
                            acc.at[pl.ds(s * _RPS, _RPS)])

        @pl.when(s == 15)
        def _():
            pltpu.sync_copy(zrow, acc.at[pl.ds(15 * _RPS, _RPS_LAST)])
        if with_counts:
            @pl.when(s == 0)
            def _():
                pltpu.sync_copy(zc, cacc)
            pltpu.sync_copy(ones, ones_v)
        plsc.subcore_barrier()

        def g_start(b, sb, j):
            pltpu.async_copy(y.at[sb.at[j]], rows[b], gsem[b])

        def g_wait(b):
            pltpu.make_async_copy(y.at[src_v[0].at[0]], rows[b],
                                  gsem[b]).wait()

        def s_start(b, db, j):
            pltpu.async_copy(rows[b], acc.at[db.at[j]], ssem[b], add=True)
            if with_counts:
                pltpu.async_copy(ones_v, cacc.at[db.at[j]], csem[b],
                                 add=True)

        def s_wait(b):
            pltpu.make_async_copy(rows[b], acc.at[dst_v[0].at[0]],
                                  ssem[b]).wait()
            if with_counts:
                pltpu.make_async_copy(ones_v, cacc.at[dst_v[0].at[0]],
                                      csem[b]).wait()

        def i_start(t_next, nsb, ndb):
            pltpu.async_copy(src_i.at[wid, t_next], nsb, isem)
            pltpu.async_copy(dst_i.at[wid, t_next], ndb, isem)

        def i_wait(nsb, ndb):
            pltpu.make_async_copy(src_i.at[wid, 0], nsb, isem).wait()
            pltpu.make_async_copy(dst_i.at[wid, 0], ndb, isem).wait()

        def phase(sb, db, nsb, ndb, t_next, pre, sw0):
            if pre is None:
                i_start(t_next, nsb, ndb)
            else:
                @pl.when(pre)
                def _():
                    i_start(t_next, nsb, ndb)

            def turn_reg(i, c2):
                for u in range(_K):
                    j = i * _K + u
                    g_wait(u)
                    s_start(u, db, j)
                    d = (u + _G) % _K
                    if u < _L and sw0 is not None:
                        @pl.when(sw0 | (i > 0))
                        def _():
                            s_wait(d)
                    else:
                        s_wait(d)
                    g_start(d, sb, j + _G)
                return c2

            lax.fori_loop(0, _NT - 1, turn_reg, 0)
            for u in range(_K):
                j = (_NT - 1) * _K + u
                g_wait(u)
                s_start(u, db, j)
                d = (u + _G) % _K
                s_wait(d)
                if u < _L:
                    g_start(d, sb, j + _G)
                else:
                    if u == _L:
                        if pre is None:
                            i_wait(nsb, ndb)
                        else:
                            @pl.when(pre)
                            def _():
                                i_wait(nsb, ndb)
                    if pre is None:
                        g_start(d, nsb, j + _G - _SCH)
                    else:
                        @pl.when(pre)
                        def _():
                            g_start(d, nsb, j + _G - _SCH)

        pltpu.sync_copy(src_i.at[wid, 0], src_v[0])
        pltpu.sync_copy(dst_i.at[wid, 0], dst_v[0])
        for b in range(_G):
            g_start(b, src_v[0], b)

        def pair(p, carry):
            phase(src_v[0], dst_v[0], src_v[1], dst_v[1],
                  2 * p + 1, None, p > 0)
            phase(src_v[1], dst_v[1], src_v[0], dst_v[0],
                  2 * p + 2, p + 1 < _ST // 2, None)
            return carry

        lax.fori_loop(0, _ST // 2, pair, 0)
        for b in range(_K - _L, _K):
            s_wait(b)
        plsc.subcore_barrier()

        @pl.when(s < 15)
        def _():
            pltpu.sync_copy(acc.at[pl.ds(s * _RPS, _RPS)],
                            parts.at[c, pl.ds(s * _RPS, _RPS)])

        @pl.when(s == 15)
        def _():
            pltpu.sync_copy(acc.at[pl.ds(15 * _RPS, _RPS_LAST)],
                            parts.at[c, pl.ds(15 * _RPS, _RPS_LAST)])
        if with_counts:
            @pl.when(s == 0)
            def _():
                pltpu.sync_copy(cacc, counts.at[c])

    if with_counts:
        def body_wc(y, src_i, dst_i, zrow, zc, ones, parts, counts,
                    src_v, dst_v, rows, acc, gsem, ssem, isem,
                    ones_v, cacc, csem):
            body(y, src_i, dst_i, zrow, zc, ones, parts, counts,
                 src_v, dst_v, rows, acc, gsem, ssem, isem,
                 ones_v, cacc, csem)
        fn = body_wc
    else:
        def body_nc(y, src_i, dst_i, zrow, zc, ones, parts,
                    src_v, dst_v, rows, acc, gsem, ssem, isem):
            body(y, src_i, dst_i, zrow, zc, ones, parts, None,
                 src_v, dst_v, rows, acc, gsem, ssem, isem)
        fn = body_nc

    return pl.kernel(fn, mesh=mesh, out_type=out_type, scratch_types=scratch)


_segsum_counts = _make_segsum(True)
_segsum = _make_segsum(False)


def _mm2_body(x_ref, wl_ref, wr_ref, b_ref, y_ref, r_ref):
    xb = x_ref[...]
    y_ref[...] = jnp.dot(xb, wl_ref[...], preferred_element_type=jnp.float32)
    r_ref[...] = (jnp.dot(xb, wr_ref[...], preferred_element_type=jnp.float32)
                  + b_ref[...])


_mm2 = pl.pallas_call(
    _mm2_body,
    grid=(_N // _BM,),
    in_specs=[
        pl.BlockSpec((_BM, _F), lambda i: (i, 0)),
        pl.BlockSpec((_F, _F), lambda i: (0, 0)),
        pl.BlockSpec((_F, _F), lambda i: (0, 0)),
        pl.BlockSpec((1, _F), lambda i: (0, 0)),
    ],
    out_specs=[pl.BlockSpec((_BM, _F), lambda i: (i, 0))] * 2,
    out_shape=[jax.ShapeDtypeStruct((_N, _F), jnp.float32)] * 2,
)


def _comb1_body(p_ref, c_ref, r1_ref, wl_ref, wr_ref, b_ref,
                y2_ref, r2_ref, inv_ref):
    p = p_ref[0] + p_ref[1]
    cnt = c_ref[0] + c_ref[1]
    inv = 1.0 / jnp.maximum(cnt, 1.0)
    h = jnp.maximum(p * inv + r1_ref[...], 0.0)
    y2_ref[...] = jnp.dot(h, wl_ref[...], preferred_element_type=jnp.float32)
    r2_ref[...] = (jnp.dot(h, wr_ref[...], preferred_element_type=jnp.float32)
                   + b_ref[...])
    inv_ref[...] = inv


_comb1 = pl.pallas_call(
    _comb1_body,
    grid=(_N // _BM,),
    in_specs=[
        pl.BlockSpec((2, _BM, _F), lambda i: (0, i, 0)),
        pl.BlockSpec((2, _BM, 1), lambda i: (0, i, 0)),
        pl.BlockSpec((_BM, _F), lambda i: (i, 0)),
        pl.BlockSpec((_F, _F), lambda i: (0, 0)),
        pl.BlockSpec((_F, _F), lambda i: (0, 0)),
        pl.BlockSpec((1, _F), lambda i: (0, 0)),
    ],
    out_specs=[
        pl.BlockSpec((_BM, _F), lambda i: (i, 0)),
        pl.BlockSpec((_BM, _F), lambda i: (i, 0)),
        pl.BlockSpec((_BM, 1), lambda i: (i, 0)),
    ],
    out_shape=[
        jax.ShapeDtypeStruct((_N, _F), jnp.float32),
        jax.ShapeDtypeStruct((_N, _F), jnp.float32),
        jax.ShapeDtypeStruct((_N, 1), jnp.float32),
    ],
)


def _comb2_body(p_ref, inv_ref, r2_ref, o_ref):
    p = p_ref[0] + p_ref[1]
    o_ref[...] = p * inv_ref[...] + r2_ref[...]


_comb2 = pl.pallas_call(
    _comb2_body,
    grid=(_N // _BM,),
    in_specs=[
        pl.BlockSpec((2, _BM, _F), lambda i: (0, i, 0)),
        pl.BlockSpec((_BM, 1), lambda i: (i, 0)),
        pl.BlockSpec((_BM, _F), lambda i: (i, 0)),
    ],
    out_specs=pl.BlockSpec((_BM, _F), lambda i: (i, 0)),
    out_shape=jax.ShapeDtypeStruct((_N, _F), jnp.float32),
)


def kernel(x, edge_index, edge_attr, W1_l, b1, W1_r, W2_l, b2, W2_r):
    del edge_attr
    src = edge_index[0].reshape(_NW, _ST, _SCH, _C).astype(jnp.int32)
    dst = edge_index[1].reshape(_NW, _ST, _SCH, _C).astype(jnp.int32)
    zrow = jnp.zeros((_RPS_LAST, _F), jnp.float32)
    zc = jnp.zeros((_N,), jnp.float32)
    ones = jnp.ones((_C,), jnp.float32)

    y1, r1 = _mm2(x, W1_l, W1_r, b1.reshape(1, _F))
    parts1, cnts = _segsum_counts(y1, src, dst, zrow, zc, ones)
    y2, r2, inv = _comb1(parts1, cnts.reshape(2, _N, 1), r1,
                         W2_l, W2_r, b2.reshape(1, _F))
    parts2, = _segsum(y2, src, dst, zrow, zc, ones)
    out = _comb2(parts2, inv, r2)
    return out


# speedup vs baseline: 1.2700x; 1.0058x over previous
import functools

import jax
import jax.numpy as jnp
from jax import lax
from jax.experimental import pallas as pl
from jax.experimental.pallas import tpu as pltpu
from jax.experimental.pallas import tpu_sc as plsc

_N = 10000
_E = 320000
_F = 128
_C = 40
_NW = 32
_NCH = _E // (_NW * _C)
_NSUB = 16
_RPS = 624
_RPS_LAST = _N - 15 * _RPS
_BM = 1000


_G = 4
_L = 1
_K = _G + _L
_ST = 10
_SCH = _NCH // _ST
_NT = _SCH // _K


def _make_segsum(with_counts: bool):
    mesh = plsc.VectorSubcoreMesh(core_axis_name="c", subcore_axis_name="s")
    out_type = [jax.ShapeDtypeStruct((2, _N, _F), jnp.float32)]
    scratch = [
        [pltpu.VMEM((_SCH, _C), jnp.int32) for _ in range(2)],
        [pltpu.VMEM((_SCH, _C), jnp.int32) for _ in range(2)],
        [pltpu.VMEM((_C, _F), jnp.float32) for _ in range(_K)],
        pltpu.VMEM_SHARED((_N, _F), jnp.float32),
        [pltpu.SemaphoreType.DMA for _ in range(_K)],
        [pltpu.SemaphoreType.DMA for _ in range(_K)],
        pltpu.SemaphoreType.DMA,
    ]
    if with_counts:
        out_type.append(jax.ShapeDtypeStruct((2, _N), jnp.float32))
        scratch += [
            pltpu.VMEM((_C,), jnp.float32),
            pltpu.VMEM_SHARED((_N,), jnp.float32),
            [pltpu.SemaphoreType.DMA for _ in range(_K)],
        ]

    def body(y, src_i, dst_i, zrow, zc, ones, parts, counts,
             src_v, dst_v, rows, acc, gsem, ssem, isem,
             ones_v=None, cacc=None, csem=None):
        c = lax.axis_index("c")
        s = lax.axis_index("s")
        wid = c * _NSUB + s

        def g_start(b, sb, j):
            pltpu.async_copy(y.at[sb.at[j]], rows[b], gsem[b])

        def g_wait(b):
            pltpu.make_async_copy(y.at[src_v[0].at[0]], rows[b],
                                  gsem[b]).wait()

        def s_start(b, db, j):
            pltpu.async_copy(rows[b], acc.at[db.at[j]], ssem[b], add=True)
            if with_counts:
                pltpu.async_copy(ones_v, cacc.at[db.at[j]], csem[b],
                                 add=True)

        def s_wait(b):
            pltpu.make_async_copy(rows[b], acc.at[dst_v[0].at[0]],
                                  ssem[b]).wait()
            if with_counts:
                pltpu.make_async_copy(ones_v, cacc.at[dst_v[0].at[0]],
                                      csem[b]).wait()

        def i_start(t_next, nsb, ndb):
            pltpu.async_copy(src_i.at[wid, t_next], nsb, isem)
            pltpu.async_copy(dst_i.at[wid, t_next], ndb, isem)

        def i_wait(nsb, ndb):
            pltpu.make_async_copy(src_i.at[wid, 0], nsb, isem).wait()
            pltpu.make_async_copy(dst_i.at[wid, 0], ndb, isem).wait()

        def phase(sb, db, nsb, ndb, t_next, pre, sw0):
            if pre is None:
                i_start(t_next, nsb, ndb)
            else:
                @pl.when(pre)
                def _():
                    i_start(t_next, nsb, ndb)

            def turn_reg(i, c2):
                for u in range(_K):
                    j = i * _K + u
                    g_wait(u)
                    s_start(u, db, j)
                    d = (u + _G) % _K
                    if u < _L and sw0 is not None:
                        @pl.when(sw0 | (i > 0))
                        def _():
                            s_wait(d)
                    else:
                        s_wait(d)
                    g_start(d, sb, j + _G)
                return c2

            lax.fori_loop(0, _NT - 1, turn_reg, 0)
            for u in range(_K):
                j = (_NT - 1) * _K + u
                g_wait(u)
                s_start(u, db, j)
                d = (u + _G) % _K
                s_wait(d)
                if u < _L:
                    g_start(d, sb, j + _G)
                else:
                    if u == _L:
                        if pre is None:
                            i_wait(nsb, ndb)
                        else:
                            @pl.when(pre)
                            def _():
                                i_wait(nsb, ndb)
                    if pre is None:
                        g_start(d, nsb, j + _G - _SCH)
                    else:
                        @pl.when(pre)
                        def _():
                            g_start(d, nsb, j + _G - _SCH)

        pltpu.sync_copy(src_i.at[wid, 0], src_v[0])
        pltpu.sync_copy(dst_i.at[wid, 0], dst_v[0])
        for b in range(_G):
            g_start(b, src_v[0], b)

        @pl.when(s < 15)
        def _():
            pltpu.sync_copy(zrow.at[pl.ds(0, _RPS)],
                            acc.at[pl.ds(s * _RPS, _RPS)])

        @pl.when(s == 15)
        def _():
            pltpu.sync_copy(zrow, acc.at[pl.ds(15 * _RPS, _RPS_LAST)])
        if with_counts:
            @pl.when(s == 0)
            def _():
                pltpu.sync_copy(zc, cacc)
            pltpu.sync_copy(ones, ones_v)
        plsc.subcore_barrier()

        def pair(p, carry):
            phase(src_v[0], dst_v[0], src_v[1], dst_v[1],
                  2 * p + 1, None, p > 0)
            phase(src_v[1], dst_v[1], src_v[0], dst_v[0],
                  2 * p + 2, p + 1 < _ST // 2, None)
            return carry

        lax.fori_loop(0, _ST // 2, pair, 0)
        for b in range(_K - _L, _K):
            s_wait(b)
        plsc.subcore_barrier()

        @pl.when(s < 15)
        def _():
            pltpu.sync_copy(acc.at[pl.ds(s * _RPS, _RPS)],
                            parts.at[c, pl.ds(s * _RPS, _RPS)])

        @pl.when(s == 15)
        def _():
            pltpu.sync_copy(acc.at[pl.ds(15 * _RPS, _RPS_LAST)],
                            parts.at[c, pl.ds(15 * _RPS, _RPS_LAST)])
        if with_counts:
            @pl.when(s == 0)
            def _():
                pltpu.sync_copy(cacc, counts.at[c])

    if with_counts:
        def body_wc(y, src_i, dst_i, zrow, zc, ones, parts, counts,
                    src_v, dst_v, rows, acc, gsem, ssem, isem,
                    ones_v, cacc, csem):
            body(y, src_i, dst_i, zrow, zc, ones, parts, counts,
                 src_v, dst_v, rows, acc, gsem, ssem, isem,
                 ones_v, cacc, csem)
        fn = body_wc
    else:
        def body_nc(y, src_i, dst_i, zrow, zc, ones, parts,
                    src_v, dst_v, rows, acc, gsem, ssem, isem):
            body(y, src_i, dst_i, zrow, zc, ones, parts, None,
                 src_v, dst_v, rows, acc, gsem, ssem, isem)
        fn = body_nc

    return pl.kernel(fn, mesh=mesh, out_type=out_type, scratch_types=scratch)


_segsum_counts = _make_segsum(True)
_segsum = _make_segsum(False)


def _mm2_body(x_ref, wl_ref, wr_ref, b_ref, y_ref, r_ref):
    xb = x_ref[...]
    y_ref[...] = jnp.dot(xb, wl_ref[...], preferred_element_type=jnp.float32)
    r_ref[...] = (jnp.dot(xb, wr_ref[...], preferred_element_type=jnp.float32)
                  + b_ref[...])


_mm2 = pl.pallas_call(
    _mm2_body,
    grid=(_N // _BM,),
    in_specs=[
        pl.BlockSpec((_BM, _F), lambda i: (i, 0)),
        pl.BlockSpec((_F, _F), lambda i: (0, 0)),
        pl.BlockSpec((_F, _F), lambda i: (0, 0)),
        pl.BlockSpec((1, _F), lambda i: (0, 0)),
    ],
    out_specs=[pl.BlockSpec((_BM, _F), lambda i: (i, 0))] * 2,
    out_shape=[jax.ShapeDtypeStruct((_N, _F), jnp.float32)] * 2,
)


def _comb1_body(p_ref, c_ref, r1_ref, wl_ref, wr_ref, b_ref,
                y2_ref, r2_ref, inv_ref):
    p = p_ref[0] + p_ref[1]
    cnt = c_ref[0] + c_ref[1]
    inv = 1.0 / jnp.maximum(cnt, 1.0)
    h = jnp.maximum(p * inv + r1_ref[...], 0.0)
    y2_ref[...] = jnp.dot(h, wl_ref[...], preferred_element_type=jnp.float32)
    r2_ref[...] = (jnp.dot(h, wr_ref[...], preferred_element_type=jnp.float32)
                   + b_ref[...])
    inv_ref[...] = inv


_comb1 = pl.pallas_call(
    _comb1_body,
    grid=(_N // _BM,),
    in_specs=[
        pl.BlockSpec((2, _BM, _F), lambda i: (0, i, 0)),
        pl.BlockSpec((2, _BM, 1), lambda i: (0, i, 0)),
        pl.BlockSpec((_BM, _F), lambda i: (i, 0)),
        pl.BlockSpec((_F, _F), lambda i: (0, 0)),
        pl.BlockSpec((_F, _F), lambda i: (0, 0)),
        pl.BlockSpec((1, _F), lambda i: (0, 0)),
    ],
    out_specs=[
        pl.BlockSpec((_BM, _F), lambda i: (i, 0)),
        pl.BlockSpec((_BM, _F), lambda i: (i, 0)),
        pl.BlockSpec((_BM, 1), lambda i: (i, 0)),
    ],
    out_shape=[
        jax.ShapeDtypeStruct((_N, _F), jnp.float32),
        jax.ShapeDtypeStruct((_N, _F), jnp.float32),
        jax.ShapeDtypeStruct((_N, 1), jnp.float32),
    ],
)


def _comb2_body(p_ref, inv_ref, r2_ref, o_ref):
    p = p_ref[0] + p_ref[1]
    o_ref[...] = p * inv_ref[...] + r2_ref[...]


_comb2 = pl.pallas_call(
    _comb2_body,
    grid=(_N // _BM,),
    in_specs=[
        pl.BlockSpec((2, _BM, _F), lambda i: (0, i, 0)),
        pl.BlockSpec((_BM, 1), lambda i: (i, 0)),
        pl.BlockSpec((_BM, _F), lambda i: (i, 0)),
    ],
    out_specs=pl.BlockSpec((_BM, _F), lambda i: (i, 0)),
    out_shape=jax.ShapeDtypeStruct((_N, _F), jnp.float32),
)


def kernel(x, edge_index, edge_attr, W1_l, b1, W1_r, W2_l, b2, W2_r):
    del edge_attr
    src = edge_index[0].reshape(_NW, _ST, _SCH, _C).astype(jnp.int32)
    dst = edge_index[1].reshape(_NW, _ST, _SCH, _C).astype(jnp.int32)
    zrow = jnp.zeros((_RPS_LAST, _F), jnp.float32)
    zc = jnp.zeros((_N,), jnp.float32)
    ones = jnp.ones((_C,), jnp.float32)

    y1, r1 = _mm2(x, W1_l, W1_r, b1.reshape(1, _F))
    parts1, cnts = _segsum_counts(y1, src, dst, zrow, zc, ones)
    y2, r2, inv = _comb1(parts1, cnts.reshape(2, _N, 1), r1,
                         W2_l, W2_r, b2.reshape(1, _F))
    parts2, = _segsum(y2, src, dst, zrow, zc, ones)
    out = _comb2(parts2, inv, r2)
    return out
